# baseline (speedup 1.0000x reference)
import functools

import jax
import jax.numpy as jnp
from jax import lax
from jax.experimental import pallas as pl
from jax.experimental.pallas import tpu as pltpu
from jax.experimental.pallas import tpu_sc as plsc

D_EMBED = 64
D_PAD = 128
SCALE = float(64 ** 0.5)

NUM_CORES = 2
NUM_SUBCORES = 16
NW = NUM_CORES * NUM_SUBCORES
LA = 128
UNROLL = 8


def _build_sc_gather(batch: int, seq: int):
    mesh = plsc.VectorSubcoreMesh(core_axis_name="c", subcore_axis_name="s")
    rows_per_w = batch // NW
    lb = seq - LA

    @functools.partial(
        pl.kernel,
        mesh=mesh,
        out_type=jax.ShapeDtypeStruct((batch, seq, D_EMBED), jnp.float32),
        scratch_types=[
            pltpu.VMEM((rows_per_w, seq), jnp.int32),
            pltpu.VMEM((2, LA, D_PAD), jnp.float32),
            pltpu.VMEM((2, lb, D_PAD), jnp.float32),
            pltpu.VMEM((LA, D_EMBED), jnp.float32),
            pltpu.VMEM((lb, D_EMBED), jnp.float32),
            [pltpu.SemaphoreType.DMA] * 2,
            [pltpu.SemaphoreType.DMA] * 2,
            pltpu.SemaphoreType.DMA,
            pltpu.SemaphoreType.DMA,
        ],
    )
    def gather_kernel(
        table_hbm, idx_hbm, out_hbm,
        idx_v, rows_a, rows_b, pack_a, pack_b, gsems_a, gsems_b, osem_a, osem_b,
    ):
        wid = lax.axis_index("s") * NUM_CORES + lax.axis_index("c")
        bb0 = wid * rows_per_w
        pltpu.sync_copy(idx_hbm.at[pl.ds(bb0, rows_per_w)], idx_v)

        def start_ga(r, j):
            pltpu.async_copy(
                table_hbm.at[idx_v.at[r, pl.ds(0, LA)]], rows_a.at[j], gsems_a[j]
            )

        def start_gb(r, j):
            pltpu.async_copy(
                table_hbm.at[idx_v.at[r, pl.ds(LA, lb)]], rows_b.at[j], gsems_b[j]
            )

        def wait_ga(j):
            pltpu.make_async_copy(
                table_hbm.at[idx_v.at[0, pl.ds(0, LA)]], rows_a.at[j], gsems_a[j]
            ).wait()

        def wait_gb(j):
            pltpu.make_async_copy(
                table_hbm.at[idx_v.at[0, pl.ds(LA, lb)]], rows_b.at[j], gsems_b[j]
            ).wait()

        def scale(rows_ref, j, pack_ref, n):
            def _scale_rows(i, c):
                r0 = i * UNROLL
                for r in range(UNROLL):
                    for q in range(D_EMBED // 16):
                        sl = pl.ds(q * 16, 16)
                        pack_ref[r0 + r, sl] = rows_ref[j, r0 + r, sl]
                return c

            lax.fori_loop(0, n // UNROLL, _scale_rows, 0)

        def start_sa(r):
            pltpu.async_copy(
                pack_a, out_hbm.at[bb0 + r, pl.ds(0, LA)], osem_a
            )

        def start_sb(r):
            pltpu.async_copy(
                pack_b, out_hbm.at[bb0 + r, pl.ds(LA, lb)], osem_b
            )

        def wait_sa():
            pltpu.make_async_copy(
                pack_a, out_hbm.at[bb0, pl.ds(0, LA)], osem_a
            ).wait()

        def wait_sb():
            pltpu.make_async_copy(
                pack_b, out_hbm.at[bb0, pl.ds(LA, lb)], osem_b
            ).wait()

        def row_body(r, j, first: bool, lookahead: bool):
            wait_ga(j)
            if lookahead:
                start_ga(r + 1, j ^ 1)
            if not first:
                wait_sa()
            scale(rows_a, j, pack_a, LA)
            start_sa(r)
            wait_gb(j)
            if lookahead:
                start_gb(r + 1, j ^ 1)
            if not first:
                wait_sb()
            scale(rows_b, j, pack_b, lb)
            start_sb(r)

        start_ga(0, 0)
        start_gb(0, 0)
        row_body(0, 0, first=True, lookahead=True)
        row_body(1, 1, first=False, lookahead=True)

        def pair(k, carry):
            row_body(k, 0, first=False, lookahead=True)
            row_body(k + 1, 1, first=False, lookahead=True)
            return carry

        lax.fori_loop(1, rows_per_w // 2 - 1, lambda q, c: pair(q * 2, c), 0)

        row_body(rows_per_w - 2, 0, first=False, lookahead=True)
        row_body(rows_per_w - 1, 1, first=False, lookahead=False)
        wait_sa()
        wait_sb()

    return gather_kernel


def kernel(inp_tokens, emb_table):
    b, s = inp_tokens.shape
    assert b % NW == 0 and s > LA
    table_pad = jnp.concatenate([emb_table * SCALE, emb_table], axis=1)
    return _build_sc_gather(b, s)(table_pad, inp_tokens)


# speedup vs baseline: 1.4099x; 1.4099x over previous
import functools

import jax
import jax.numpy as jnp
from jax import lax
from jax.experimental import pallas as pl
from jax.experimental.pallas import tpu as pltpu
from jax.experimental.pallas import tpu_sc as plsc

D_EMBED = 64
D_PAD = 128
SCALE = float(64 ** 0.5)

NUM_CORES = 2
NUM_SUBCORES = 16
NW = NUM_CORES * NUM_SUBCORES
LA = 128
UNROLL = 8


def _build_sc_gather(batch: int, seq: int):
    mesh = plsc.VectorSubcoreMesh(core_axis_name="c", subcore_axis_name="s")
    rows_per_w = batch // NW
    lb = seq - LA

    @functools.partial(
        pl.kernel,
        mesh=mesh,
        out_type=jax.ShapeDtypeStruct((batch, seq, D_EMBED), jnp.float32),
        scratch_types=[
            pltpu.VMEM((rows_per_w, seq), jnp.int32),
            pltpu.VMEM((2, LA, D_PAD), jnp.float32),
            pltpu.VMEM((2, lb, D_PAD), jnp.float32),
            pltpu.VMEM((2, LA, D_EMBED), jnp.float32),
            pltpu.VMEM((lb, D_EMBED), jnp.float32),
            [pltpu.SemaphoreType.DMA] * 2,
            [pltpu.SemaphoreType.DMA] * 2,
            [pltpu.SemaphoreType.DMA] * 2,
            pltpu.SemaphoreType.DMA,
        ],
    )
    def gather_kernel(
        table_hbm, idx_hbm, out_hbm,
        idx_v, rows_a, rows_b, pack_a, pack_b, gsems_a, gsems_b, osems_a, osem_b,
    ):
        wid = lax.axis_index("s") * NUM_CORES + lax.axis_index("c")
        bb0 = wid * rows_per_w
        pltpu.sync_copy(idx_hbm.at[pl.ds(bb0, rows_per_w)], idx_v)

        def start_ga(r, j):
            pltpu.async_copy(
                table_hbm.at[idx_v.at[r, pl.ds(0, LA)]], rows_a.at[j], gsems_a[j]
            )

        def start_gb(r, j):
            pltpu.async_copy(
                table_hbm.at[idx_v.at[r, pl.ds(LA, lb)]], rows_b.at[j], gsems_b[j]
            )

        def wait_ga(j):
            pltpu.make_async_copy(
                table_hbm.at[idx_v.at[0, pl.ds(0, LA)]], rows_a.at[j], gsems_a[j]
            ).wait()

        def wait_gb(j):
            pltpu.make_async_copy(
                table_hbm.at[idx_v.at[0, pl.ds(LA, lb)]], rows_b.at[j], gsems_b[j]
            ).wait()

        def scale(rows_ref, j, pack_ref, n):
            def _scale_rows(i, c):
                r0 = i * UNROLL
                for r in range(UNROLL):
                    for q in range(D_EMBED // 16):
                        sl = pl.ds(q * 16, 16)
                        pack_ref[r0 + r, sl] = rows_ref[j, r0 + r, sl] * SCALE
                return c

            lax.fori_loop(0, n // UNROLL, _scale_rows, 0)

        def start_sa(r, j):
            pltpu.async_copy(
                pack_a.at[j], out_hbm.at[bb0 + r, pl.ds(0, LA)], osems_a[j]
            )

        def start_sb(r):
            pltpu.async_copy(
                pack_b, out_hbm.at[bb0 + r, pl.ds(LA, lb)], osem_b
            )

        def wait_sa(j):
            pltpu.make_async_copy(
                pack_a.at[j], out_hbm.at[bb0, pl.ds(0, LA)], osems_a[j]
            ).wait()

        def wait_sb():
            pltpu.make_async_copy(
                pack_b, out_hbm.at[bb0, pl.ds(LA, lb)], osem_b
            ).wait()

        def row_body(r, j, first_a: bool, first_b: bool, lookahead: bool):
            wait_ga(j)
            if lookahead:
                start_ga(r + 1, j ^ 1)
            if not first_a:
                wait_sa(j)
            scale(rows_a, j, pack_a.at[j], LA)
            start_sa(r, j)
            wait_gb(j)
            if lookahead:
                start_gb(r + 1, j ^ 1)
            if not first_b:
                wait_sb()
            scale(rows_b, j, pack_b, lb)
            start_sb(r)

        start_ga(0, 0)
        start_gb(0, 0)
        row_body(0, 0, first_a=True, first_b=True, lookahead=True)
        row_body(1, 1, first_a=True, first_b=False, lookahead=True)

        def pair(k, carry):
            row_body(k, 0, first_a=False, first_b=False, lookahead=True)
            row_body(k + 1, 1, first_a=False, first_b=False, lookahead=True)
            return carry

        lax.fori_loop(1, rows_per_w // 2 - 1, lambda q, c: pair(q * 2, c), 0)

        row_body(rows_per_w - 2, 0, first_a=False, first_b=False, lookahead=True)
        row_body(rows_per_w - 1, 1, first_a=False, first_b=False, lookahead=False)
        wait_sa(0)
        wait_sa(1)
        wait_sb()

    return gather_kernel


def kernel(inp_tokens, emb_table):
    b, s = inp_tokens.shape
    assert b % NW == 0 and s > LA
    table_pad = jnp.pad(emb_table, ((0, 0), (0, D_PAD - D_EMBED)))
    return _build_sc_gather(b, s)(table_pad, inp_tokens)


# speedup vs baseline: 1.5609x; 1.1071x over previous
import functools

import jax
import jax.numpy as jnp
from jax import lax
from jax.experimental import pallas as pl
from jax.experimental.pallas import tpu as pltpu
from jax.experimental.pallas import tpu_sc as plsc

D_EMBED = 64
D_PAD = 128
SCALE = float(64 ** 0.5)

NUM_CORES = 2
NUM_SUBCORES = 16
NW = NUM_CORES * NUM_SUBCORES
CHUNK = 128
NBUF = 4
NOBUF = 2
UNROLL = 8


def _build_sc_gather(n_chunks: int):
    mesh = plsc.VectorSubcoreMesh(core_axis_name="c", subcore_axis_name="s")
    total = NW * n_chunks * CHUNK

    @functools.partial(
        pl.kernel,
        mesh=mesh,
        out_type=jax.ShapeDtypeStruct((total, D_EMBED), jnp.float32),
        scratch_types=[
            pltpu.VMEM((n_chunks, CHUNK), jnp.int32),
            pltpu.VMEM((NBUF, CHUNK, D_PAD), jnp.float32),
            pltpu.VMEM((NOBUF, CHUNK, D_EMBED), jnp.float32),
            [pltpu.SemaphoreType.DMA] * NBUF,
            [pltpu.SemaphoreType.DMA] * NOBUF,
        ],
    )
    def gather_kernel(
        table_hbm, idx_hbm, out_hbm, idx_v, rows_v, pack_v, gsems, osems
    ):
        wid = lax.axis_index("s") * NUM_CORES + lax.axis_index("c")
        base = wid * (n_chunks * CHUNK)
        pltpu.sync_copy(idx_hbm.at[pl.ds(wid * n_chunks, n_chunks)], idx_v)

        def start_gather(g, b):
            pltpu.async_copy(table_hbm.at[idx_v.at[g]], rows_v.at[b], gsems[b])

        def wait_gather(b):
            pltpu.make_async_copy(
                table_hbm.at[idx_v.at[0]], rows_v.at[b], gsems[b]
            ).wait()

        def scale(b, o):
            def _scale_rows(i, c):
                r0 = i * UNROLL
                for r in range(UNROLL):
                    for q in range(D_EMBED // 16):
                        sl = pl.ds(q * 16, 16)
                        pack_v[o, r0 + r, sl] = rows_v[b, r0 + r, sl] * SCALE
                return c

            lax.fori_loop(0, CHUNK // UNROLL, _scale_rows, 0)

        def start_store(g, o):
            pltpu.async_copy(
                pack_v.at[o], out_hbm.at[pl.ds(base + g * CHUNK, CHUNK)], osems[o]
            )

        def wait_store(o):
            pltpu.make_async_copy(
                pack_v.at[o], out_hbm.at[pl.ds(base, CHUNK)], osems[o]
            ).wait()

        def chunk_body(g, b, o, wait_out: bool, lookahead: bool):
            wait_gather(b)
            if wait_out:
                wait_store(o)
            scale(b, o)
            start_store(g, o)
            if lookahead:
                start_gather(g + 2, (b + 2) % NBUF)

        start_gather(0, 0)
        start_gather(1, 1)
        chunk_body(0, 0, 0, wait_out=False, lookahead=True)
        chunk_body(1, 1, 1, wait_out=False, lookahead=True)
        chunk_body(2, 2, 0, wait_out=True, lookahead=True)
        chunk_body(3, 3, 1, wait_out=True, lookahead=True)

        def quad(k, carry):
            for b in range(NBUF):
                chunk_body(k + b, b, b % NOBUF, wait_out=True, lookahead=True)
            return carry

        lax.fori_loop(1, n_chunks // NBUF - 1, lambda q, c: quad(q * NBUF, c), 0)

        e = n_chunks - NBUF
        chunk_body(e + 0, 0, 0, wait_out=True, lookahead=True)
        chunk_body(e + 1, 1, 1, wait_out=True, lookahead=True)
        chunk_body(e + 2, 2, 0, wait_out=True, lookahead=False)
        chunk_body(e + 3, 3, 1, wait_out=True, lookahead=False)
        for o in range(NOBUF):
            wait_store(o)

    return gather_kernel


def kernel(inp_tokens, emb_table):
    b, s = inp_tokens.shape
    total = b * s
    assert total % (NW * CHUNK) == 0
    n_chunks = total // (NW * CHUNK)
    table_pad = jnp.pad(emb_table, ((0, 0), (0, D_PAD - D_EMBED)))
    idx = inp_tokens.reshape(total // CHUNK, CHUNK)
    out = _build_sc_gather(n_chunks)(table_pad, idx)
    return out.reshape(b, s, D_EMBED)
